# Initial kernel scaffold; baseline (speedup 1.0000x reference)
import jax
import jax.numpy as jnp
from jax.experimental import pallas as pl


def kernel(x, edge_index, edge_attr, batch, W_pre, b_pre, Wl1, bl1, Wr1, We1, be1, Wl2, bl2, Wr2, We2, be2, Wl3, bl3, Wr3, We3, be3, W_hh1, b_hh1, W_hh2, b_hh2, W_oo, b_oo, W_oh, b_oh, W_h1, b_h1, gamma_h, beta_h):
    raise NotImplementedError("write your pallas kernel here")


# speedup vs baseline: 1.5073x; 1.5073x over previous
import functools

import jax
import jax.numpy as jnp
from jax import lax
from jax.experimental import pallas as pl
from jax.experimental.pallas import tpu as pltpu
from jax.experimental.pallas import tpu_sc as plsc

_N = 10000
_E = 320000
_H = 128
_G = 64
_NC = 2
_NS = 16
_NW = _NC * _NS
_EPW = _E // _NW
_C = 80
_NCH = _EPW // _C
_NP = 10240
_RPT = _NP // _NS
_NZ = _RPT // _C

_f32 = jnp.float32
_i32 = jnp.int32


def _edge_compute(rows, ea_ref, we_ref, n_edges):
    def body(g, _):
        eag = ea_ref[pl.ds(g * 16, 16)]
        for l in range(16):
            eav = eag[jnp.full((16,), l, _i32)]
            e = g * 16 + l
            for j in range(8):
                sl = pl.ds(j * 16, 16)
                rows[e, sl] = jnp.maximum(rows[e, sl] + eav * we_ref[sl], 0.0)
        return 0
    lax.fori_loop(0, n_edges // 16, body, 0)


def _conv_body(h, src, dst, ea, we, agg_o,
               agg_sh, we_v, src_v, dst_v, ea_v, rows_v, gsem):
    cid = lax.axis_index("c")
    sid = lax.axis_index("s")
    wid = cid * _NS + sid

    def zrow(i, _):
        for j in range(8):
            rows_v[i, pl.ds(j * 16, 16)] = jnp.zeros((16,), _f32)
        return 0
    lax.fori_loop(0, _C, zrow, 0)

    row0 = sid * _RPT
    for k in range(_NZ):
        pltpu.sync_copy(rows_v, agg_sh.at[pl.ds(row0 + k * _C, _C)])
    pltpu.sync_copy(we, we_v)
    plsc.subcore_barrier()

    base = wid * _EPW

    def chunk(c, _):
        off = base + c * _C
        pltpu.sync_copy(src.at[pl.ds(off, _C)], src_v)
        pltpu.sync_copy(dst.at[pl.ds(off, _C)], dst_v)
        pltpu.sync_copy(ea.at[pl.ds(off, _C)], ea_v)
        pltpu.async_copy(h.at[src_v], rows_v, gsem).wait()
        _edge_compute(rows_v, ea_v, we_v, _C)
        pltpu.sync_copy(rows_v, agg_sh.at[dst_v], add=True)
        return 0
    lax.fori_loop(0, _NCH, chunk, 0)

    plsc.subcore_barrier()
    pltpu.sync_copy(agg_sh.at[pl.ds(row0, _RPT)], agg_o.at[cid, pl.ds(row0, _RPT)])


@functools.lru_cache(maxsize=None)
def _sc_calls():
    sc_mesh = plsc.VectorSubcoreMesh(core_axis_name="c", subcore_axis_name="s",
                                     num_cores=_NC, num_subcores=_NS)
    conv_call = pl.kernel(
        _conv_body,
        out_type=jax.ShapeDtypeStruct((_NC, _NP, _H), _f32),
        mesh=sc_mesh,
        scratch_types=[
            pltpu.VMEM_SHARED((_NP, _H), _f32),
            pltpu.VMEM((_H,), _f32),
            pltpu.VMEM((_C,), _i32),
            pltpu.VMEM((_C,), _i32),
            pltpu.VMEM((_C,), _f32),
            pltpu.VMEM((_C, _H), _f32),
            pltpu.SemaphoreType.DMA,
        ],
    )
    return conv_call


_R = 1000
_NBLK = _N // _R

_leaky = lambda v: jnp.where(v >= 0, v, 0.01 * v)


def _split(a):
    hi = a.astype(jnp.bfloat16)
    lo = (a - hi.astype(_f32)).astype(jnp.bfloat16)
    return hi, lo


def _dot3(a, b, dims):
    ah, al = _split(a)
    bh, bl = _split(b)
    dg = lambda p, q: lax.dot_general(p, q, (dims, ((), ())),
                                      preferred_element_type=_f32)
    return dg(ah, bh) + (dg(ah, bl) + dg(al, bh))


def _dotT(a, w):
    return _dot3(a, w, ((1,), (1,)))


def _tc1_body(x_ref, wp, bp, be, h0_ref, g1_ref):
    hh = jnp.maximum(_dotT(x_ref[...], wp[...]) + bp[...], 0.0)
    h0_ref[...] = hh
    g1_ref[...] = hh + be[...]


def _tcmid_body(a0, a1, c0, c1, hp, wl, bl, wr, whh, bhh, be, hm_ref, g_ref):
    cnt = c0[:, :1] + c1[:, :1]
    invc = 1.0 / jnp.maximum(cnt, 1.0)
    a = (a0[...] + a1[...]) * invc
    hc = jnp.maximum(_dotT(a, wl[...]) + bl[...] + _dotT(hp[...], wr[...]), 0.0)
    hm = _leaky(_dotT(hc, whh[...]) + bhh[...])
    hm_ref[...] = hm
    g_ref[...] = hm + be[...]


def _tc4_body(a0, a1, c0, c1, hp, wl, bl, wr, woo, boo, bt, s_ref, c_ref):
    cnt = c0[:, :1] + c1[:, :1]
    invc = 1.0 / jnp.maximum(cnt, 1.0)
    a = (a0[...] + a1[...]) * invc
    hc = jnp.maximum(_dotT(a, wl[...]) + bl[...] + _dotT(hp[...], wr[...]), 0.0)
    hf = _leaky(_dotT(hc, woo[...]) + boo[...])
    onehot = (bt[...] == lax.broadcasted_iota(_i32, (_R, _G), 1)).astype(_f32)
    s = _dot3(onehot, hf, ((0,), (0,)))
    c = lax.dot_general(onehot, jnp.ones_like(hf), (((0,), (0,)), ((), ())),
                        preferred_element_type=_f32)

    @pl.when(pl.program_id(0) == 0)
    def _():
        s_ref[...] = jnp.zeros_like(s_ref)
        c_ref[...] = jnp.zeros_like(c_ref)

    s_ref[...] += s
    c_ref[...] += c


def _tc5_body(s, c, woh, boh, wh1, bh1, gam, bet, o_ref):
    p = s[...] / jnp.maximum(c[...], 1.0)
    p = _dotT(p, woh[...]) + boh[...]
    mu = jnp.mean(p, axis=0, keepdims=True)
    d = p - mu
    var = jnp.mean(d * d, axis=0, keepdims=True)
    pn = d * lax.rsqrt(var + 1e-5) * gam[...] + bet[...]
    pn = _leaky(pn)
    o = jnp.sum(pn * wh1[...], axis=1, keepdims=True) + bh1[...]
    o_ref[...] = jnp.maximum(o, 0.0)


def _row_spec():
    return pl.BlockSpec((_R, _H), lambda i: (i, 0))


def _cnt_spec():
    return pl.BlockSpec((_R, _H), lambda i: (i, 0))


_w_spec = pl.BlockSpec((_H, _H), lambda i: (0, 0))
_b_spec = pl.BlockSpec((1, _H), lambda i: (0, 0))

_tc1 = pl.pallas_call(
    _tc1_body, grid=(_NBLK,),
    in_specs=[_row_spec(), _w_spec, _b_spec, _b_spec],
    out_specs=[_row_spec(), _row_spec()],
    out_shape=[jax.ShapeDtypeStruct((_N, _H), _f32)] * 2,
)

_tcmid = pl.pallas_call(
    _tcmid_body, grid=(_NBLK,),
    in_specs=[_row_spec(), _row_spec(), _cnt_spec(), _cnt_spec(), _row_spec(),
              _w_spec, _b_spec, _w_spec, _w_spec, _b_spec, _b_spec],
    out_specs=[_row_spec(), _row_spec()],
    out_shape=[jax.ShapeDtypeStruct((_N, _H), _f32)] * 2,
)

_tc4 = pl.pallas_call(
    _tc4_body, grid=(_NBLK,),
    in_specs=[_row_spec(), _row_spec(), _cnt_spec(), _cnt_spec(), _row_spec(),
              _w_spec, _b_spec, _w_spec, _w_spec, _b_spec,
              pl.BlockSpec((_R, 1), lambda i: (i, 0))],
    out_specs=[pl.BlockSpec((_G, _H), lambda i: (0, 0))] * 2,
    out_shape=[jax.ShapeDtypeStruct((_G, _H), _f32)] * 2,
)

_tc5 = pl.pallas_call(
    _tc5_body, grid=(1,),
    in_specs=[pl.BlockSpec((_G, _H), lambda i: (0, 0)),
              pl.BlockSpec((_G, _H), lambda i: (0, 0)),
              _w_spec, _b_spec, _b_spec,
              pl.BlockSpec((1, 1), lambda i: (0, 0)),
              _b_spec, _b_spec],
    out_specs=pl.BlockSpec((_G, 1), lambda i: (0, 0)),
    out_shape=jax.ShapeDtypeStruct((_G, 1), _f32),
)


def kernel(x, edge_index, edge_attr, batch, W_pre, b_pre, Wl1, bl1, Wr1, We1, be1,
           Wl2, bl2, Wr2, We2, be2, Wl3, bl3, Wr3, We3, be3,
           W_hh1, b_hh1, W_hh2, b_hh2, W_oo, b_oo, W_oh, b_oh, W_h1, b_h1,
           gamma_h, beta_h):
    src = edge_index[0].astype(_i32)
    dst = edge_index[1].astype(_i32)
    ea = edge_attr[:, 0].astype(_f32)
    bt = batch.reshape(_N, 1).astype(_i32)
    r2 = lambda v: v.reshape(1, -1).astype(_f32)
    conv_call = _sc_calls()

    h0, g1 = _tc1(x, W_pre, r2(b_pre), r2(be1))
    cnt = conv_call(jnp.ones((_N, _H), _f32), src, dst, ea, jnp.zeros((_H,), _f32))
    agg1 = conv_call(g1, src, dst, ea, We1[:, 0])
    hm1, g2 = _tcmid(agg1[0], agg1[1], cnt[0], cnt[1], h0,
                     Wl1, r2(bl1), Wr1, W_hh1, r2(b_hh1), r2(be2))
    agg2 = conv_call(g2, src, dst, ea, We2[:, 0])
    hm2, g3 = _tcmid(agg2[0], agg2[1], cnt[0], cnt[1], hm1,
                     Wl2, r2(bl2), Wr2, W_hh2, r2(b_hh2), r2(be3))
    agg3 = conv_call(g3, src, dst, ea, We3[:, 0])
    s, c = _tc4(agg3[0], agg3[1], cnt[0], cnt[1], hm2,
                Wl3, r2(bl3), Wr3, W_oo, r2(b_oo), bt)
    out = _tc5(s, c, W_oh, r2(b_oh), W_h1, b_h1.reshape(1, 1),
               r2(gamma_h), r2(beta_h))
    return out
